# baseline (speedup 1.0000x reference)
import functools

import jax
import jax.numpy as jnp
import numpy as np
from jax import lax
from jax.experimental import pallas as pl
from jax.experimental.pallas import tpu as pltpu


def _conv(x, w, b, stride, pad):
    y = lax.conv_general_dilated(x, w, (stride, stride), ((pad, pad), (pad, pad)),
                                 dimension_numbers=('NCHW', 'OIHW', 'NCHW'))
    return y + b[None, :, None, None]


def _resblock(x, w1, b1, w2, b2):
    h = jax.nn.relu(x)
    h = _conv(h, w1, b1, 1, 1)
    h = jax.nn.relu(h)
    h = _conv(h, w2, b2, 1, 0)
    return x + h


def _encoder(x, p):
    x = jax.nn.relu(_conv(x, p['e_c1_w'], p['e_c1_b'], 2, 1))
    x = jax.nn.relu(_conv(x, p['e_c2_w'], p['e_c2_b'], 2, 1))
    x = _conv(x, p['e_c3_w'], p['e_c3_b'], 1, 1)
    for i in range(2):
        x = _resblock(x, p['e_rb%d_w1' % i], p['e_rb%d_b1' % i],
                      p['e_rb%d_w2' % i], p['e_rb%d_b2' % i])
    x = jax.nn.relu(x)
    x = _conv(x, p['e_out_w'], p['e_out_b'], 1, 0)
    return x


_NUM_CODES = 512


def _vq_body(x_ref, embt_ref, emb_ref, out_ref):
    x = x_ref[...]
    embt = embt_ref[...]
    m = jax.lax.dot_general(x.astype(jnp.bfloat16), embt.astype(jnp.bfloat16),
                            (((1,), (0,)), ((), ())),
                            preferred_element_type=jnp.float32)
    sx = jnp.sum(x * x, axis=1, keepdims=True)
    se = jnp.sum(embt * embt, axis=0, keepdims=True)
    d2 = (sx + se) - 2.0 * m
    dis = jnp.sqrt(jnp.maximum(d2, 0.0))
    minv = jnp.min(dis, axis=1, keepdims=True)
    iota = jax.lax.broadcasted_iota(jnp.int32, dis.shape, 1)
    idx = jnp.min(jnp.where(dis == minv, iota, _NUM_CODES), axis=1)
    onehot = (iota == idx[:, None]).astype(jnp.float32)
    q = jax.lax.dot_general(onehot, emb_ref[...], (((1,), (0,)), ((), ())),
                            precision=jax.lax.Precision.HIGHEST,
                            preferred_element_type=jnp.float32)
    qx = x + (q - x)
    out_ref[...] = qx.astype(jnp.bfloat16)


@functools.partial(jax.jit, static_argnames=('tile_m',))
def _vq_pallas(xf, emb, tile_m=512):
    M, C = xf.shape
    K = emb.shape[0]
    embt = emb.T
    return pl.pallas_call(
        _vq_body,
        grid=(M // tile_m,),
        in_specs=[
            pl.BlockSpec((tile_m, C), lambda i: (i, 0)),
            pl.BlockSpec((C, K), lambda i: (0, 0)),
            pl.BlockSpec((K, C), lambda i: (0, 0)),
        ],
        out_specs=pl.BlockSpec((tile_m, C), lambda i: (i, 0)),
        out_shape=jax.ShapeDtypeStruct((M, C), jnp.bfloat16),
    )(xf, embt, emb)


_P56 = 58
_N56 = _P56 * _P56
_P112 = 114
_N112 = _P112 * _P112

_OFF3 = tuple((dy - 1) * _P56 + (dx - 1) for dy in range(3) for dx in range(3))
_OFF3_T1 = tuple(iy * _P56 + ix for iy in (-1, 0, 1) for ix in (-1, 0, 1))
_OFF3_T2 = tuple(jy * _P112 + jx for jy in (-1, 0, 1) for jx in (-1, 0, 1))


def _shift_rows(p, s):
    if s == 0:
        return p
    n = p.shape[0]
    z = jnp.zeros((abs(s), p.shape[1]), p.dtype)
    if s > 0:
        return jnp.concatenate([p[s:], z], axis=0)
    return jnp.concatenate([z, p[:n + s]], axis=0)


def _im2col(x, offsets):
    cols = [_shift_rows(x, s) for s in offsets]
    return jnp.concatenate(cols, axis=1).astype(jnp.bfloat16)


def _mm(a, b):
    return jax.lax.dot_general(a, b, (((1,), (0,)), ((), ())),
                               preferred_element_type=jnp.float32)


def _dec_trunk_body(x_ref, wc1_ref, bc1_ref,
                    rw1a_ref, rb1a_ref, rw2a_ref, rb2a_ref,
                    rw1b_ref, rb1b_ref, rw2b_ref, rb2b_ref,
                    wt1_ref, bt1_ref, mask_ref, out_ref):
    mask = mask_ref[...]
    x0 = x_ref[0].astype(jnp.float32)
    x1 = (_mm(_im2col(x0, _OFF3), wc1_ref[...]) + bc1_ref[...]) * mask
    for w1_ref, b1_ref, w2_ref, b2_ref in (
            (rw1a_ref, rb1a_ref, rw2a_ref, rb2a_ref),
            (rw1b_ref, rb1b_ref, rw2b_ref, rb2b_ref)):
        h = jax.nn.relu(x1)
        h = (_mm(_im2col(h, _OFF3), w1_ref[...]) + b1_ref[...]) * mask
        h = jax.nn.relu(h)
        h = _mm(h.astype(jnp.bfloat16), w2_ref[...]) + b2_ref[...]
        x1 = x1 + h * mask
    x1 = jax.nn.relu(x1)
    y = _mm(_im2col(x1, _OFF3_T1), wt1_ref[...]) + bt1_ref[...]
    y = jax.nn.relu(y) * mask
    out_ref[0] = y.astype(jnp.bfloat16)


def _dec_t2_body(x_ref, w_ref, b_ref, mask_ref, out_ref):
    x0 = x_ref[0].astype(jnp.float32)
    y = _mm(_im2col(x0, _OFF3_T2), w_ref[...]) + b_ref[...]
    out_ref[0] = y * mask_ref[...]


def _full_spec(shape):
    nd = len(shape)
    return pl.BlockSpec(shape, lambda i, _nd=nd: (0,) * _nd)


@jax.jit
def _dec_trunk_pallas(qpad, wc1, bc1, rw1a, rb1a, rw2a, rb2a,
                      rw1b, rb1b, rw2b, rb2b, wt1, bt1, mask):
    args = [qpad, wc1, bc1, rw1a, rb1a, rw2a, rb2a,
            rw1b, rb1b, rw2b, rb2b, wt1, bt1, mask]
    in_specs = [pl.BlockSpec((1, _N56, 64), lambda i: (i, 0, 0))]
    in_specs += [_full_spec(a.shape) for a in args[1:]]
    return pl.pallas_call(
        _dec_trunk_body,
        grid=(qpad.shape[0],),
        in_specs=in_specs,
        out_specs=pl.BlockSpec((1, _N56, 256), lambda i: (i, 0, 0)),
        out_shape=jax.ShapeDtypeStruct((qpad.shape[0], _N56, 256), jnp.bfloat16),
    )(*args)


@jax.jit
def _dec_t2_pallas(xpad, w2, b2, mask):
    return pl.pallas_call(
        _dec_t2_body,
        grid=(xpad.shape[0],),
        in_specs=[pl.BlockSpec((1, _N112, 64), lambda i: (i, 0, 0)),
                  _full_spec(w2.shape), _full_spec(b2.shape),
                  _full_spec(mask.shape)],
        out_specs=pl.BlockSpec((1, _N112, 12), lambda i: (i, 0, 0)),
        out_shape=jax.ShapeDtypeStruct((xpad.shape[0], _N112, 12), jnp.float32),
    )(xpad, w2, b2, mask)


def _taps3x3(w):
    co, ci = w.shape[0], w.shape[1]
    return w.transpose(2, 3, 1, 0).reshape(9 * ci, co).astype(jnp.bfloat16)


_CT_K = {(-1, 0): 3, (0, 0): 1, (0, 1): 2, (1, 1): 0}


def _t_weights(w):
    ci, co = w.shape[0], w.shape[1]
    zero = jnp.zeros((ci, co), w.dtype)
    blocks = []
    for iy in (-1, 0, 1):
        for ix in (-1, 0, 1):
            phases = []
            for py in (0, 1):
                for px in (0, 1):
                    ky = _CT_K.get((iy, py))
                    kx = _CT_K.get((ix, px))
                    phases.append(zero if ky is None or kx is None
                                  else w[:, :, ky, kx])
            blocks.append(jnp.stack(phases, axis=1))
    out = jnp.stack(blocks, axis=0)
    return out.reshape(9 * ci, 4 * co).astype(jnp.bfloat16)


def _mask_plane(side, pitch):
    m = np.zeros((pitch, pitch, 1), np.float32)
    m[1:side + 1, 1:side + 1] = 1.0
    return jnp.asarray(m.reshape(pitch * pitch, 1))


_MASK56 = _mask_plane(56, _P56)
_MASK112 = _mask_plane(112, _P112)


def kernel(img, params):
    p = params
    x = _encoder(img, p)
    B, C, H, W = x.shape
    xf = x.transpose(0, 2, 3, 1).reshape(-1, C)
    qx = _vq_pallas(xf, p['embedding'])

    qpad = jnp.pad(qx.reshape(B, H, W, C), ((0, 0), (1, 1), (1, 1), (0, 0)))
    qpad = qpad.reshape(B, _N56, C)

    wc1 = _taps3x3(p['d_c1_w'])
    bc1 = p['d_c1_b'][None, :]
    rws = []
    for i in range(2):
        rws += [_taps3x3(p['d_rb%d_w1' % i]), p['d_rb%d_b1' % i][None, :],
                p['d_rb%d_w2' % i][:, :, 0, 0].T.astype(jnp.bfloat16),
                p['d_rb%d_b2' % i][None, :]]
    wt1 = _t_weights(p['d_t1_w'])
    bt1 = jnp.tile(p['d_t1_b'], 4)[None, :]

    ya = _dec_trunk_pallas(qpad, wc1, bc1, *rws, wt1, bt1, _MASK56)

    yb = ya.reshape(B, _P56, _P56, 2, 2, 64)[:, 1:57, 1:57]
    yb = yb.transpose(0, 1, 3, 2, 4, 5).reshape(B, 112, 112, 64)
    yb = jnp.pad(yb, ((0, 0), (1, 1), (1, 1), (0, 0))).reshape(B, _N112, 64)

    w2 = _t_weights(p['d_t2_w'])
    b2 = jnp.tile(p['d_t2_b'], 4)[None, :]
    yc = _dec_t2_pallas(yb, w2, b2, _MASK112)

    pred = yc.reshape(B, _P112, _P112, 2, 2, 3)[:, 1:113, 1:113]
    pred = pred.transpose(0, 5, 1, 3, 2, 4).reshape(B, 3, 224, 224)
    return pred


# speedup vs baseline: 1.4481x; 1.4481x over previous
import functools

import jax
import jax.numpy as jnp
import numpy as np
from jax import lax
from jax.experimental import pallas as pl
from jax.experimental.pallas import tpu as pltpu


def _conv(x, w, b, stride, pad):
    y = lax.conv_general_dilated(x, w, (stride, stride), ((pad, pad), (pad, pad)),
                                 dimension_numbers=('NCHW', 'OIHW', 'NCHW'))
    return y + b[None, :, None, None]


def _resblock(x, w1, b1, w2, b2):
    h = jax.nn.relu(x)
    h = _conv(h, w1, b1, 1, 1)
    h = jax.nn.relu(h)
    h = _conv(h, w2, b2, 1, 0)
    return x + h


def _encoder(x, p):
    x = jax.nn.relu(_conv(x, p['e_c1_w'], p['e_c1_b'], 2, 1))
    x = jax.nn.relu(_conv(x, p['e_c2_w'], p['e_c2_b'], 2, 1))
    x = _conv(x, p['e_c3_w'], p['e_c3_b'], 1, 1)
    for i in range(2):
        x = _resblock(x, p['e_rb%d_w1' % i], p['e_rb%d_b1' % i],
                      p['e_rb%d_w2' % i], p['e_rb%d_b2' % i])
    x = jax.nn.relu(x)
    x = _conv(x, p['e_out_w'], p['e_out_b'], 1, 0)
    return x


_NUM_CODES = 512
_P56 = 58
_N56 = _P56 * _P56
_HW = 56 * 56

_OFF3 = tuple((dy - 1) * _P56 + (dx - 1) for dy in range(3) for dx in range(3))
_OFF3_T = tuple(iy * _P56 + ix for iy in (-1, 0, 1) for ix in (-1, 0, 1))


def _shift_rows(p, s):
    if s == 0:
        return p
    n = p.shape[0]
    z = jnp.zeros((abs(s), p.shape[1]), p.dtype)
    if s > 0:
        return jnp.concatenate([p[s:], z], axis=0)
    return jnp.concatenate([z, p[:n + s]], axis=0)


def _im2col(x, offsets):
    xb = x.astype(jnp.bfloat16)
    return jnp.concatenate([_shift_rows(xb, s) for s in offsets], axis=1)


def _mm(a, b):
    return jax.lax.dot_general(a, b, (((1,), (0,)), ((), ())),
                               preferred_element_type=jnp.float32)


def _fused_body(x_ref, embt_ref, emb3_ref,
                wc1_ref, bc1_ref,
                rw1a_ref, rb1a_ref, rw2a_ref, rb2a_ref,
                rw1b_ref, rb1b_ref, rw2b_ref, rb2b_ref,
                wt1_ref, bt1_ref, wt2_ref, bt2_ref, mask_ref,
                out_ref, pad_ref):
    mask = mask_ref[...]
    x = x_ref[0]

    embt = embt_ref[...]
    m = _mm(x.astype(jnp.bfloat16), embt.astype(jnp.bfloat16))
    sx = jnp.sum(x * x, axis=1, keepdims=True)
    se = jnp.sum(embt * embt, axis=0, keepdims=True)
    d2 = (sx + se) - 2.0 * m
    dis = jnp.sqrt(jnp.maximum(d2, 0.0))
    minv = jnp.min(dis, axis=1, keepdims=True)
    iota = jax.lax.broadcasted_iota(jnp.int32, dis.shape, 1)
    idx = jnp.min(jnp.where(dis == minv, iota, _NUM_CODES), axis=1)
    onehot = (iota == idx[:, None]).astype(jnp.bfloat16)
    g = _mm(onehot, emb3_ref[...])
    q = (g[:, 0:64] + g[:, 64:128]) + g[:, 128:192]
    qx = x + (q - x)

    pad_ref[...] = jnp.zeros((_N56, 64), jnp.float32)
    for y in range(56):
        pad_ref[pl.ds((y + 1) * _P56 + 1, 56), :] = qx[y * 56:(y + 1) * 56, :]
    x0 = pad_ref[...]

    x1 = (_mm(_im2col(x0, _OFF3), wc1_ref[...]) + bc1_ref[...]) * mask
    for w1_ref, b1_ref, w2_ref, b2_ref in (
            (rw1a_ref, rb1a_ref, rw2a_ref, rb2a_ref),
            (rw1b_ref, rb1b_ref, rw2b_ref, rb2b_ref)):
        h = jax.nn.relu(x1)
        h = (_mm(_im2col(h, _OFF3), w1_ref[...]) + b1_ref[...]) * mask
        h = jax.nn.relu(h)
        h = _mm(h.astype(jnp.bfloat16), w2_ref[...]) + b2_ref[...]
        x1 = x1 + h * mask
    x1 = jax.nn.relu(x1)

    y = _mm(_im2col(x1, _OFF3_T), wt1_ref[...]) + bt1_ref[...]
    y = jax.nn.relu(y) * mask

    out = _mm(_im2col(y, _OFF3_T), wt2_ref[...]) + bt2_ref[...]
    out_ref[0] = out * mask


@jax.jit
def _fused_pallas(xf, embt, emb3, wc1, bc1, rws, wt1, bt1, wt2, bt2, mask):
    (rw1a, rb1a, rw2a, rb2a), (rw1b, rb1b, rw2b, rb2b) = rws
    args = [xf, embt, emb3, wc1, bc1, rw1a, rb1a, rw2a, rb2a,
            rw1b, rb1b, rw2b, rb2b, wt1, bt1, wt2, bt2, mask]
    in_specs = [pl.BlockSpec((1, _HW, 64), lambda i: (i, 0, 0))]
    for a in args[1:]:
        nd = len(a.shape)
        in_specs.append(pl.BlockSpec(a.shape, lambda i, _nd=nd: (0,) * _nd))
    return pl.pallas_call(
        _fused_body,
        grid=(xf.shape[0],),
        in_specs=in_specs,
        out_specs=pl.BlockSpec((1, _N56, 48), lambda i: (i, 0, 0)),
        out_shape=jax.ShapeDtypeStruct((xf.shape[0], _N56, 48), jnp.float32),
        scratch_shapes=[pltpu.VMEM((_N56, 64), jnp.float32)],
    )(*args)


def _taps3x3(w):
    co, ci = w.shape[0], w.shape[1]
    return w.transpose(2, 3, 1, 0).reshape(9 * ci, co).astype(jnp.bfloat16)


_CT_TAPS = {0: ((-1, 3), (0, 1)), 1: ((0, 2), (1, 0))}


def _t1_weights(w):
    ci, co = w.shape[0], w.shape[1]
    zero = jnp.zeros((ci, co), w.dtype)
    ymap = {(py, iy): ky for py in (0, 1) for iy, ky in _CT_TAPS[py]}
    blocks = []
    for iy in (-1, 0, 1):
        for ix in (-1, 0, 1):
            phases = []
            for py in (0, 1):
                for px in (0, 1):
                    ky = ymap.get((py, iy))
                    kx = ymap.get((px, ix))
                    phases.append(zero if ky is None or kx is None
                                  else w[:, :, ky, kx])
            blocks.append(jnp.stack(phases, axis=1))
    out = jnp.stack(blocks, axis=0)
    return out.reshape(9 * ci, 4 * co).astype(jnp.bfloat16)


def _t2_weights(w):
    ci, co = w.shape[0], w.shape[1]
    ymap = {}
    for py in (0, 1):
        for qy in (0, 1):
            for j, ky in _CT_TAPS[qy]:
                src = py + j
                ymap.setdefault((src // 2, src % 2), {})[(py, qy)] = ky
    zero = jnp.zeros((ci, co), w.dtype)
    blocks = []
    for sy in (-1, 0, 1):
        for sx in (-1, 0, 1):
            for psy in (0, 1):
                for psx in (0, 1):
                    my = ymap.get((sy, psy), {})
                    mx = ymap.get((sx, psx), {})
                    cols = []
                    for py in (0, 1):
                        for qy in (0, 1):
                            for px in (0, 1):
                                for qx in (0, 1):
                                    ky = my.get((py, qy))
                                    kx = mx.get((px, qx))
                                    cols.append(zero if ky is None or kx is None
                                                else w[:, :, ky, kx])
                    blocks.append(jnp.stack(cols, axis=1))
    out = jnp.stack(blocks, axis=0)
    return out.reshape(9 * 4 * ci, 16 * co).astype(jnp.bfloat16)


def _emb_split3(emb):
    hi = emb.astype(jnp.bfloat16)
    r1 = emb - hi.astype(jnp.float32)
    mid = r1.astype(jnp.bfloat16)
    lo = (r1 - mid.astype(jnp.float32)).astype(jnp.bfloat16)
    return jnp.concatenate([hi, mid, lo], axis=1)


def _mask_plane():
    m = np.zeros((_P56, _P56, 1), np.float32)
    m[1:57, 1:57] = 1.0
    return m.reshape(_N56, 1)


_MASK56 = _mask_plane()


def kernel(img, params):
    p = params
    x = _encoder(img, p)
    B, C, H, W = x.shape
    xf = x.transpose(0, 2, 3, 1).reshape(B, H * W, C)

    emb = p['embedding']
    rws = tuple((_taps3x3(p['d_rb%d_w1' % i]), p['d_rb%d_b1' % i][None, :],
                 p['d_rb%d_w2' % i][:, :, 0, 0].T.astype(jnp.bfloat16),
                 p['d_rb%d_b2' % i][None, :]) for i in range(2))
    out = _fused_pallas(
        xf, emb.T, _emb_split3(emb),
        _taps3x3(p['d_c1_w']), p['d_c1_b'][None, :], rws,
        _t1_weights(p['d_t1_w']), jnp.tile(p['d_t1_b'], 4)[None, :],
        _t2_weights(p['d_t2_w']), jnp.tile(p['d_t2_b'], 16)[None, :],
        jnp.asarray(_MASK56))

    pred = out.reshape(B, _P56, _P56, 4, 4, 3)[:, 1:57, 1:57]
    pred = pred.transpose(0, 5, 1, 3, 2, 4).reshape(B, 3, 224, 224)
    return pred
